# speedup vs baseline: 3451.5632x; 3451.5632x over previous
import jax
import jax.numpy as jnp
from jax import lax
from jax.experimental import pallas as pl
from jax.experimental.pallas import tpu as pltpu

_MIN_DIST = 4
_PRIMES = (2.0, 3.0, 5.0, 7.0)
_PAIR_PRODUCTS = (14.0, 15.0, 35.0)


def _greedy_body(conr_ref, cont_ref, seqr_ref, seqc_ref, out_ref,
                 work, fcol, frow):
    L = conr_ref.shape[0]
    ii = lax.broadcasted_iota(jnp.int32, (L, L), 0)
    jj = lax.broadcasted_iota(jnp.int32, (L, L), 1)

    sym = (conr_ref[...] + cont_ref[...]) * 0.5
    band = (jj - ii >= _MIN_DIST) | (ii - jj >= _MIN_DIST)

    sr = seqr_ref[...]
    m4r = jnp.max(sr, axis=0, keepdims=True)
    pr = jnp.full((1, L), _PRIMES[3], jnp.float32)
    pr = jnp.where(sr[2:3, :] == m4r, _PRIMES[2], pr)
    pr = jnp.where(sr[1:2, :] == m4r, _PRIMES[1], pr)
    pr = jnp.where(sr[0:1, :] == m4r, _PRIMES[0], pr)
    sc = seqc_ref[...]
    m4c = jnp.max(sc, axis=1, keepdims=True)
    pc = jnp.full((L, 1), _PRIMES[3], jnp.float32)
    pc = jnp.where(sc[:, 2:3] == m4c, _PRIMES[2], pc)
    pc = jnp.where(sc[:, 1:2] == m4c, _PRIMES[1], pc)
    pc = jnp.where(sc[:, 0:1] == m4c, _PRIMES[0], pc)
    pp = pc * pr
    pm = (pp == _PAIR_PRODUCTS[0]) | (pp == _PAIR_PRODUCTS[1]) \
        | (pp == _PAIR_PRODUCTS[2])
    pm = pm | (m4r < 1.0) | (m4c < 1.0)

    con2 = jnp.where(band & pm, sym, 0.0)
    work[...] = con2
    out_ref[...] = jnp.zeros((L, L), jnp.float32)
    fcol[...] = jnp.ones((L, 1), jnp.float32)
    frow[...] = jnp.ones((1, L), jnp.float32)

    def cond(c):
        step, changed = c
        return (changed > 0) & (step < L)

    def body(c):
        step, _ = c
        E = work[...] * fcol[...] * frow[...]
        mr = jnp.max(E, axis=1, keepdims=True)
        ar = jnp.min(jnp.where(E == mr, jj, L), axis=1, keepdims=True)
        mc = jnp.max(E, axis=0, keepdims=True)
        ac = jnp.min(jnp.where(E == mc, ii, L), axis=0, keepdims=True)
        mutual = (jj == ar) & (ii == ac) & (mr > 0.0)
        out_ref[...] = jnp.where(mutual, work[...], out_ref[...])
        mf = mutual.astype(jnp.float32)
        mrow = jnp.max(mf, axis=1, keepdims=True)
        mcol = jnp.max(mf, axis=0, keepdims=True)
        fcol[...] = fcol[...] * (1.0 - mrow)
        frow[...] = frow[...] * (1.0 - mcol)
        changed = (jnp.sum(mrow) > 0.0).astype(jnp.int32)
        return (step + jnp.int32(1), changed)

    lax.while_loop(cond, body, (jnp.int32(0), jnp.int32(1)))


def _run(con2d, con2dT, seqr, seqc, interpret=False):
    L = con2d.shape[0]
    return pl.pallas_call(
        _greedy_body,
        out_shape=jax.ShapeDtypeStruct((L, L), jnp.float32),
        scratch_shapes=[
            pltpu.VMEM((L, L), jnp.float32),
            pltpu.VMEM((L, 1), jnp.float32),
            pltpu.VMEM((1, L), jnp.float32),
        ],
        interpret=interpret,
    )(con2d, con2dT, seqr, seqc)


def kernel(con, feat):
    shape = con.shape
    L = shape[-1]
    con2d = con.reshape(L, L)
    con2dT = jnp.swapaxes(con2d, 0, 1)
    seqr = feat.reshape(feat.shape[1], L, L)[:4, :, 0]
    seqc = jnp.swapaxes(seqr, 0, 1)
    out = _run(con2d, con2dT, seqr, seqc)
    return out.reshape(shape)
